# Initial kernel scaffold; baseline (speedup 1.0000x reference)
import jax
import jax.numpy as jnp
from jax.experimental import pallas as pl


def kernel(x, edge_index, edge_attr, batch, x_emb1, x_emb2, x_emb3, edge_emb1, edge_emb2, W1, b1, W2, b2, ln_g, ln_b):
    raise NotImplementedError("write your pallas kernel here")


# speedup vs baseline: 9.0069x; 9.0069x over previous
import functools

import jax
import jax.numpy as jnp
from jax import lax
from jax.experimental import pallas as pl
from jax.experimental.pallas import tpu as pltpu
from jax.experimental.pallas import tpu_sc as plsc

NC = 2
NS = 16
NW = NC * NS
K = 128
DUMP = 64
F32 = jnp.float32


def _scatter_body(h_hbm, srcix, dstix, out, acc, sbuf, dbuf, stage, zbuf,
                  gsem, *, n, npad, wins):
    core = lax.axis_index("c")
    sub = lax.axis_index("s")
    wid = core * NS + sub
    rows = npad // NS
    zrows = zbuf.shape[0]

    def zb(i, c):
        zbuf[i // 8, pl.ds((i % 8) * 16, 16)] = jnp.zeros((16,), F32)
        return c
    lax.fori_loop(0, (zrows * zbuf.shape[1]) // 16, zb, 0)

    @pl.when(jnp.logical_and(core == 0, sub < NS - 1))
    def _():
        pltpu.sync_copy(h_hbm.at[pl.ds(sub * rows, rows)],
                        acc.at[pl.ds(sub * rows, rows)])

    @pl.when(jnp.logical_and(core == 0, sub == NS - 1))
    def _():
        tail = n - (NS - 1) * rows
        pltpu.sync_copy(h_hbm.at[pl.ds(sub * rows, tail)],
                        acc.at[pl.ds(sub * rows, tail)])
        for q in range((npad - n) // zrows):
            pltpu.sync_copy(zbuf, acc.at[pl.ds(n + q * zrows, zrows)])

    @pl.when(core != 0)
    def _():
        for q in range(rows // zrows):
            pltpu.sync_copy(zbuf, acc.at[pl.ds(sub * rows + q * zrows, zrows)])

    plsc.subcore_barrier()

    def win(j, c):
        off = (wid * wins + j) * K
        pltpu.sync_copy(srcix.at[pl.ds(off, K)], sbuf.at[0])
        pltpu.sync_copy(dstix.at[pl.ds(off, K)], dbuf.at[0])
        pltpu.async_copy(h_hbm.at[sbuf.at[0]], stage, gsem).wait()
        pltpu.sync_copy(stage, acc.at[dbuf.at[0]], add=True)
        return c
    lax.fori_loop(0, wins, win, 0)

    plsc.subcore_barrier()
    pltpu.sync_copy(acc.at[pl.ds(sub * rows, rows)],
                    out.at[core, pl.ds(sub * rows, rows)])


def _make_scatter_kernel(n, npad, emb, wins):
    mesh = plsc.VectorSubcoreMesh(core_axis_name="c", subcore_axis_name="s")
    return pl.kernel(
        functools.partial(_scatter_body, n=n, npad=npad, wins=wins),
        out_type=jax.ShapeDtypeStruct((NC, npad, emb), F32),
        mesh=mesh,
        scratch_types=[
            pltpu.VMEM_SHARED((npad, emb), F32),
            pltpu.VMEM((1, K), jnp.int32),
            pltpu.VMEM((1, K), jnp.int32),
            pltpu.VMEM((K, emb), F32),
            pltpu.VMEM((80, emb), F32),
            pltpu.SemaphoreType.DMA,
        ],
    )


def _cnt_body(elemix, out, acc, ibuf, ones, zbuf, *, n, wins):
    core = lax.axis_index("c")
    sub = lax.axis_index("s")
    wid = core * NS + sub
    zn = zbuf.shape[0]

    def zb(i, c):
        zbuf[pl.ds(i * 16, 16)] = jnp.zeros((16,), F32)
        return c
    lax.fori_loop(0, zn // 16, zb, 0)
    for q in range(8):
        ones[pl.ds(q * 16, 16)] = jnp.ones((16,), F32)

    words = (n * 128) // NS
    def zc(q, c):
        pltpu.sync_copy(zbuf, acc.at[pl.ds(sub * words + q * zn, zn)])
        return c
    lax.fori_loop(0, words // zn, zc, 0)
    plsc.subcore_barrier()

    def win(j, c):
        off = (wid * wins + j) * K
        pltpu.sync_copy(elemix.at[pl.ds(off, K)], ibuf.at[0])
        pltpu.sync_copy(ones, acc.at[ibuf.at[0]], add=True)
        return c
    lax.fori_loop(0, wins, win, 0)

    plsc.subcore_barrier()
    pltpu.sync_copy(acc.at[pl.ds(sub * words, words)],
                    out.at[core, pl.ds(sub * words, words)])


def _make_cnt_kernel(n, wins):
    mesh = plsc.VectorSubcoreMesh(core_axis_name="c", subcore_axis_name="s")
    return pl.kernel(
        functools.partial(_cnt_body, n=n, wins=wins),
        out_type=jax.ShapeDtypeStruct((NC, n * 128), F32),
        mesh=mesh,
        scratch_types=[
            pltpu.VMEM_SHARED((n * 128,), F32),
            pltpu.VMEM((1, K), jnp.int32),
            pltpu.VMEM((K,), F32),
            pltpu.VMEM((16000,), F32),
        ],
    )


_PREC = None


def _emb_body(x0, x1, x2, tab, out):
    it = lax.broadcasted_iota(jnp.int32, (x0.shape[0], 128), 1)
    c0 = (x0[...] == it) & (it < 16)
    c1 = (x1[...] == it - 16) & (it >= 16) & (it < 32)
    c2 = (x2[...] == it - 32) & (it >= 32) & (it < 48)
    oh = jnp.where(c0 | c1 | c2, 1.0, 0.0)
    out[...] = jnp.dot(oh, tab[...], preferred_element_type=F32,
                       precision=_PREC)


def _emb_lookup(x, x_emb1, x_emb2, x_emb3, n, emb):
    tab = jnp.concatenate([
        jnp.pad(x_emb1, ((0, 16 - x_emb1.shape[0]), (0, 0))),
        jnp.pad(x_emb2, ((0, 16 - x_emb2.shape[0]), (0, 0))),
        jnp.pad(x_emb3, ((0, 16 - x_emb3.shape[0]), (0, 0))),
        jnp.zeros((80, emb), F32),
    ], axis=0)
    b = 1000
    grid = n // b
    return pl.pallas_call(
        _emb_body,
        grid=(grid,),
        in_specs=[
            pl.BlockSpec((b, 1), lambda i: (i, 0)),
            pl.BlockSpec((b, 1), lambda i: (i, 0)),
            pl.BlockSpec((b, 1), lambda i: (i, 0)),
            pl.BlockSpec((128, emb), lambda i: (0, 0)),
        ],
        out_specs=pl.BlockSpec((b, emb), lambda i: (i, 0)),
        out_shape=jax.ShapeDtypeStruct((n, emb), F32),
    )(x[:, 0:1], x[:, 1:2], x[:, 2:3], tab)


def _mlp_body(p0, p1, c0, c1, ce, w1, b1, w2, b2, g, b, out, *, last):
    cnt = c0[...] + c1[...]
    agg = p0[...] + p1[...] + jnp.dot(cnt, ce[...], preferred_element_type=F32,
                                      precision=_PREC)
    z = jnp.maximum(
        jnp.dot(agg, w1[...], preferred_element_type=F32, precision=_PREC)
        + b1[...], 0.0)
    h2 = (jnp.dot(z, w2[...], preferred_element_type=F32, precision=_PREC)
          + b2[...])
    mu = jnp.mean(h2, axis=-1, keepdims=True)
    var = jnp.mean((h2 - mu) ** 2, axis=-1, keepdims=True)
    hn = (h2 - mu) / jnp.sqrt(var + 1e-5) * g[...] + b[...]
    if not last:
        hn = jnp.maximum(hn, 0.0)
    out[...] = hn


def _mlp_layer(p0, p1, c0, c1, ce, w1, b1, w2, b2, g, b, *, last, n, emb):
    blk = 1000
    grid = n // blk
    full = lambda r, c: pl.BlockSpec((r, c), lambda i: (0, 0))
    row = lambda c: pl.BlockSpec((blk, c), lambda i: (i, 0))
    return pl.pallas_call(
        functools.partial(_mlp_body, last=last),
        grid=(grid,),
        in_specs=[
            row(emb), row(emb), row(128), row(128),
            full(128, emb), full(emb, 2 * emb), full(1, 2 * emb),
            full(2 * emb, emb), full(1, emb), full(1, emb), full(1, emb),
        ],
        out_specs=row(emb),
        out_shape=jax.ShapeDtypeStruct((n, emb), F32),
    )(p0, p1, c0, c1, ce, w1, b1[None], w2, b2[None], g[None], b[None])


def kernel(x, edge_index, edge_attr, batch, x_emb1, x_emb2, x_emb3,
           edge_emb1, edge_emb2, W1, b1, W2, b2, ln_g, ln_b):
    n = x.shape[0]
    e = edge_index.shape[1]
    emb = x_emb1.shape[1]
    nl = W1.shape[0]
    i32 = jnp.int32

    src = edge_index[0]
    dst = edge_index[1]
    combo = edge_attr[:, 0] * 3 + edge_attr[:, 1]

    wins = -(-e // (NW * K))
    e_pad = wins * NW * K
    epad = e_pad - e
    pad_src = jnp.arange(epad, dtype=i32) % n
    pad_dst = n + jnp.arange(epad, dtype=i32) % DUMP
    src_p = jnp.concatenate([src, pad_src])
    dst_p = jnp.concatenate([dst, pad_dst])

    elem = dst * 128 + combo
    elem_self = jnp.arange(n, dtype=i32) * 128 + 12
    ne = e + n
    ewins = -(-ne // (NW * K))
    nepad = ewins * NW * K - ne
    pad_elem = (jnp.arange(nepad, dtype=i32) % n) * 128 + 127
    elem_p = jnp.concatenate([elem, elem_self, pad_elem])

    ce = (edge_emb1[:, :, None, :] + edge_emb2[:, None, :, :]).reshape(
        nl, 18, emb)
    ce_pad = jnp.zeros((nl, 128, emb), F32).at[:, :18].set(ce)

    h = _emb_lookup(x, x_emb1, x_emb2, x_emb3, n, emb)

    cnt2 = _make_cnt_kernel(n, ewins)(elem_p).reshape(NC, n, 128)
    npad = NS * 80 * (-(-n // (NS * 80)))
    scat = _make_scatter_kernel(n, npad, emb, wins)

    for l in range(nl):
        parts = scat(h, src_p, dst_p)
        h = _mlp_layer(parts[0], parts[1], cnt2[0], cnt2[1], ce_pad[l],
                       W1[l], b1[l], W2[l], b2[l], ln_g[l], ln_b[l],
                       last=(l == nl - 1), n=n, emb=emb)
    return h


# speedup vs baseline: 16.3897x; 1.8197x over previous
import functools

import jax
import jax.numpy as jnp
from jax import lax
from jax.experimental import pallas as pl
from jax.experimental.pallas import tpu as pltpu
from jax.experimental.pallas import tpu_sc as plsc

NC = 2
NS = 16
NW = NC * NS
K = 128
DUMP = 64
F32 = jnp.float32


CHW = 16
CROWS = 2 * CHW


def _scatter_body(h_hbm, edges, zeros, out, acc, ibuf, stage, gsems, isems,
                  *, n, npad, wins):
    core = lax.axis_index("c")
    sub = lax.axis_index("s")
    wid = core * NS + sub
    rows = npad // NS
    nch = wins // CHW
    ebase = wid * wins * 2

    for p in range(2):
        pltpu.async_copy(edges.at[pl.ds(ebase + p * CROWS, CROWS)],
                         ibuf.at[p], isems.at[p])

    @pl.when(jnp.logical_and(core == 0, sub < NS - 1))
    def _():
        pltpu.sync_copy(h_hbm.at[pl.ds(sub * rows, rows)],
                        acc.at[pl.ds(sub * rows, rows)])

    @pl.when(jnp.logical_and(core == 0, sub == NS - 1))
    def _():
        tail = n - (NS - 1) * rows
        pltpu.sync_copy(h_hbm.at[pl.ds(sub * rows, tail)],
                        acc.at[pl.ds(sub * rows, tail)])
        pltpu.sync_copy(zeros.at[pl.ds(0, npad - n)],
                        acc.at[pl.ds(n, npad - n)])

    @pl.when(core != 0)
    def _():
        pltpu.sync_copy(zeros, acc.at[pl.ds(sub * rows, rows)])

    plsc.subcore_barrier()

    def chunk(cj, c):
        p = cj % 2
        pltpu.make_async_copy(edges.at[pl.ds(0, CROWS)], ibuf.at[p],
                              isems.at[p]).wait()
        for w in range(2):
            pltpu.async_copy(h_hbm.at[ibuf.at[p, 2 * w]], stage.at[w],
                             gsems.at[w])
        for w in range(CHW):
            b = w % 2
            pltpu.make_async_copy(h_hbm.at[pl.ds(0, K)], stage.at[b],
                                  gsems.at[b]).wait()
            pltpu.sync_copy(stage.at[b], acc.at[ibuf.at[p, 2 * w + 1]],
                            add=True)
            if w + 2 < CHW:
                pltpu.async_copy(h_hbm.at[ibuf.at[p, 2 * (w + 2)]],
                                 stage.at[b], gsems.at[b])

        @pl.when(cj + 2 < nch)
        def _():
            pltpu.async_copy(
                edges.at[pl.ds(ebase + (cj + 2) * CROWS, CROWS)],
                ibuf.at[p], isems.at[p])
        return c
    lax.fori_loop(0, nch, chunk, 0)

    plsc.subcore_barrier()
    pltpu.sync_copy(acc.at[pl.ds(sub * rows, rows)],
                    out.at[core, pl.ds(sub * rows, rows)])


def _make_scatter_kernel(n, npad, emb, wins):
    mesh = plsc.VectorSubcoreMesh(core_axis_name="c", subcore_axis_name="s")
    return pl.kernel(
        functools.partial(_scatter_body, n=n, npad=npad, wins=wins),
        out_type=jax.ShapeDtypeStruct((NC, npad, emb), F32),
        mesh=mesh,
        scratch_types=[
            pltpu.VMEM_SHARED((npad, emb), F32),
            pltpu.VMEM((2, CROWS, K), jnp.int32),
            pltpu.VMEM((2, K, emb), F32),
            pltpu.SemaphoreType.DMA((2,)),
            pltpu.SemaphoreType.DMA((2,)),
        ],
    )


def _cnt_body(elemix, out, acc, ibuf, ones, zbuf, *, n, wins):
    core = lax.axis_index("c")
    sub = lax.axis_index("s")
    wid = core * NS + sub
    zn = zbuf.shape[0]

    def zb(i, c):
        zbuf[pl.ds(i * 16, 16)] = jnp.zeros((16,), F32)
        return c
    lax.fori_loop(0, zn // 16, zb, 0)
    for q in range(8):
        ones[pl.ds(q * 16, 16)] = jnp.ones((16,), F32)

    words = (n * 128) // NS
    def zc(q, c):
        pltpu.sync_copy(zbuf, acc.at[pl.ds(sub * words + q * zn, zn)])
        return c
    lax.fori_loop(0, words // zn, zc, 0)
    plsc.subcore_barrier()

    def win(j, c):
        off = (wid * wins + j) * K
        pltpu.sync_copy(elemix.at[pl.ds(off, K)], ibuf.at[0])
        pltpu.sync_copy(ones, acc.at[ibuf.at[0]], add=True)
        return c
    lax.fori_loop(0, wins, win, 0)

    plsc.subcore_barrier()
    pltpu.sync_copy(acc.at[pl.ds(sub * words, words)],
                    out.at[core, pl.ds(sub * words, words)])


def _make_cnt_kernel(n, wins):
    mesh = plsc.VectorSubcoreMesh(core_axis_name="c", subcore_axis_name="s")
    return pl.kernel(
        functools.partial(_cnt_body, n=n, wins=wins),
        out_type=jax.ShapeDtypeStruct((NC, n * 128), F32),
        mesh=mesh,
        scratch_types=[
            pltpu.VMEM_SHARED((n * 128,), F32),
            pltpu.VMEM((1, K), jnp.int32),
            pltpu.VMEM((K,), F32),
            pltpu.VMEM((16000,), F32),
        ],
    )


_PREC = None


def _emb_body(x0, x1, x2, tab, out):
    it = lax.broadcasted_iota(jnp.int32, (x0.shape[0], 128), 1)
    c0 = (x0[...] == it) & (it < 16)
    c1 = (x1[...] == it - 16) & (it >= 16) & (it < 32)
    c2 = (x2[...] == it - 32) & (it >= 32) & (it < 48)
    oh = jnp.where(c0 | c1 | c2, 1.0, 0.0)
    out[...] = jnp.dot(oh, tab[...], preferred_element_type=F32,
                       precision=_PREC)


def _emb_lookup(x, x_emb1, x_emb2, x_emb3, n, emb):
    tab = jnp.concatenate([
        jnp.pad(x_emb1, ((0, 16 - x_emb1.shape[0]), (0, 0))),
        jnp.pad(x_emb2, ((0, 16 - x_emb2.shape[0]), (0, 0))),
        jnp.pad(x_emb3, ((0, 16 - x_emb3.shape[0]), (0, 0))),
        jnp.zeros((80, emb), F32),
    ], axis=0)
    b = 1000
    grid = n // b
    return pl.pallas_call(
        _emb_body,
        grid=(grid,),
        in_specs=[
            pl.BlockSpec((b, 1), lambda i: (i, 0)),
            pl.BlockSpec((b, 1), lambda i: (i, 0)),
            pl.BlockSpec((b, 1), lambda i: (i, 0)),
            pl.BlockSpec((128, emb), lambda i: (0, 0)),
        ],
        out_specs=pl.BlockSpec((b, emb), lambda i: (i, 0)),
        out_shape=jax.ShapeDtypeStruct((n, emb), F32),
    )(x[:, 0:1], x[:, 1:2], x[:, 2:3], tab)


def _mlp_body(p0, p1, c0, c1, ce, w1, b1, w2, b2, g, b, out, *, last):
    cnt = c0[...] + c1[...]
    agg = p0[...] + p1[...] + jnp.dot(cnt, ce[...], preferred_element_type=F32,
                                      precision=_PREC)
    z = jnp.maximum(
        jnp.dot(agg, w1[...], preferred_element_type=F32, precision=_PREC)
        + b1[...], 0.0)
    h2 = (jnp.dot(z, w2[...], preferred_element_type=F32, precision=_PREC)
          + b2[...])
    mu = jnp.mean(h2, axis=-1, keepdims=True)
    var = jnp.mean((h2 - mu) ** 2, axis=-1, keepdims=True)
    hn = (h2 - mu) / jnp.sqrt(var + 1e-5) * g[...] + b[...]
    if not last:
        hn = jnp.maximum(hn, 0.0)
    out[...] = hn


def _mlp_layer(p0, p1, c0, c1, ce, w1, b1, w2, b2, g, b, *, last, n, emb):
    blk = 1000
    grid = n // blk
    full = lambda r, c: pl.BlockSpec((r, c), lambda i: (0, 0))
    row = lambda c: pl.BlockSpec((blk, c), lambda i: (i, 0))
    return pl.pallas_call(
        functools.partial(_mlp_body, last=last),
        grid=(grid,),
        in_specs=[
            row(emb), row(emb), row(128), row(128),
            full(128, emb), full(emb, 2 * emb), full(1, 2 * emb),
            full(2 * emb, emb), full(1, emb), full(1, emb), full(1, emb),
        ],
        out_specs=row(emb),
        out_shape=jax.ShapeDtypeStruct((n, emb), F32),
    )(p0, p1, c0, c1, ce, w1, b1[None], w2, b2[None], g[None], b[None])


def kernel(x, edge_index, edge_attr, batch, x_emb1, x_emb2, x_emb3,
           edge_emb1, edge_emb2, W1, b1, W2, b2, ln_g, ln_b):
    n = x.shape[0]
    e = edge_index.shape[1]
    emb = x_emb1.shape[1]
    nl = W1.shape[0]
    i32 = jnp.int32

    src = edge_index[0]
    dst = edge_index[1]
    combo = edge_attr[:, 0] * 3 + edge_attr[:, 1]

    wins = CHW * (-(-e // (NW * K * CHW)))
    e_pad = wins * NW * K
    epad = e_pad - e
    pad_src = jnp.arange(epad, dtype=i32) % n
    pad_dst = n + jnp.arange(epad, dtype=i32) % DUMP
    src_p = jnp.concatenate([src, pad_src]).reshape(NW * wins, K)
    dst_p = jnp.concatenate([dst, pad_dst]).reshape(NW * wins, K)
    edges_il = jnp.stack([src_p, dst_p], axis=1).reshape(NW * wins * 2, K)

    elem = dst * 128 + combo
    elem_self = jnp.arange(n, dtype=i32) * 128 + 12
    ne = e + n
    ewins = -(-ne // (NW * K))
    nepad = ewins * NW * K - ne
    pad_elem = (jnp.arange(nepad, dtype=i32) % n) * 128 + 127
    elem_p = jnp.concatenate([elem, elem_self, pad_elem])

    ce = (edge_emb1[:, :, None, :] + edge_emb2[:, None, :, :]).reshape(
        nl, 18, emb)
    ce_pad = jnp.zeros((nl, 128, emb), F32).at[:, :18].set(ce)

    h = _emb_lookup(x, x_emb1, x_emb2, x_emb3, n, emb)

    cnt2 = _make_cnt_kernel(n, ewins)(elem_p).reshape(NC, n, 128)
    npad = NS * 80 * (-(-n // (NS * 80)))
    zeros = jnp.zeros((npad // NS, emb), F32)
    scat = _make_scatter_kernel(n, npad, emb, wins)

    for l in range(nl):
        parts = scat(h, edges_il, zeros)
        h = _mlp_layer(parts[0], parts[1], cnt2[0], cnt2[1], ce_pad[l],
                       W1[l], b1[l], W2[l], b2[l], ln_g[l], ln_b[l],
                       last=(l == nl - 1), n=n, emb=emb)
    return h
